# baseline (speedup 1.0000x reference)
import functools

import jax
import jax.numpy as jnp
from jax import lax
from jax.experimental import pallas as pl
from jax.experimental.pallas import tpu as pltpu, tpu_sc as plsc

D_MODEL = 512
MAX_LEN = 200
VOCAB = 100
BATCH = 1024

NC, NS, LANES = 2, 16, 16
NW = NC * NS
ROWS_PER_W = BATCH // NW
TOK_PER_W = ROWS_PER_W * MAX_LEN
TPOS = 8
GB = 8
TILE = TPOS * GB
NCHUNK = MAX_LEN // TPOS
NGROUP = ROWS_PER_W // GB
NTILE = NCHUNK * NGROUP
DJ = D_MODEL // LANES
NBUF = 3


def _positional_encoding():
    even_i = jnp.arange(0, D_MODEL, 2).astype(jnp.float32)
    denominator = jnp.power(10000.0, even_i / D_MODEL)
    position = jnp.arange(MAX_LEN, dtype=jnp.float32).reshape(MAX_LEN, 1)
    even = jnp.sin(position / denominator)
    odd = jnp.cos(position / denominator)
    stacked = jnp.stack([even, odd], axis=2)
    return stacked.reshape(MAX_LEN, D_MODEL)


@functools.partial(
    pl.kernel,
    mesh=plsc.VectorSubcoreMesh(core_axis_name="c", subcore_axis_name="s"),
    out_type=jax.ShapeDtypeStruct((BATCH * MAX_LEN, D_MODEL), jnp.float32),
    scratch_types=[
        pltpu.VMEM((TOK_PER_W,), jnp.int32),
        pltpu.VMEM((TPOS, D_MODEL), jnp.float32),
        pltpu.VMEM((NBUF, TILE), jnp.int32),
        pltpu.VMEM((TILE, D_MODEL), jnp.float32),
        pltpu.VMEM((TILE, D_MODEL), jnp.float32),
        pltpu.VMEM((TILE, D_MODEL), jnp.float32),
        pltpu.SemaphoreType.DMA,
        pltpu.SemaphoreType.DMA,
        pltpu.SemaphoreType.DMA,
        pltpu.SemaphoreType.DMA,
        pltpu.SemaphoreType.DMA,
        pltpu.SemaphoreType.DMA,
    ],
    compiler_params=pltpu.CompilerParams(needs_layout_passes=False),
)
def _sc_embed(x_hbm, table_hbm, pos_hbm, out_hbm,
              xw_v, posc_v, idx_v, rows0_v, rows1_v, rows2_v,
              g_sem0, g_sem1, g_sem2, w_sem0, w_sem1, w_sem2):
    wid = lax.axis_index("s") * NC + lax.axis_index("c")
    base_tok = wid * TOK_PER_W
    pltpu.sync_copy(x_hbm.at[pl.ds(base_tok, TOK_PER_W)], xw_v)
    lane = lax.iota(jnp.int32, LANES)

    rows_bufs = (rows0_v, rows1_v, rows2_v)
    g_sems = (g_sem0, g_sem1, g_sem2)
    w_sems = (w_sem0, w_sem1, w_sem2)

    def launch_gather(t, slot):
        ci = t >> 2
        g = t & (NGROUP - 1)
        c = ci * TPOS
        for k in range(TILE // LANES):
            gl = lane + (LANES * k)
            bb = gl >> 3
            tt = gl & (TPOS - 1)
            src = (g * GB + bb) * MAX_LEN + c + tt
            idx_v[slot, pl.ds(LANES * k, LANES)] = (
                plsc.load_gather(xw_v, [src]) + wid * VOCAB)
        pltpu.async_copy(table_hbm.at[idx_v.at[slot]], rows_bufs[slot],
                         g_sems[slot])

    def wait_gather(slot):
        pltpu.make_async_copy(out_hbm.at[pl.ds(0, TILE), :], rows_bufs[slot],
                              g_sems[slot]).wait()

    def drain_writes(slot):
        pltpu.make_async_copy(rows_bufs[slot], out_hbm.at[pl.ds(0, TILE), :],
                              w_sems[slot]).wait()

    def add_pos(slot):
        rows_v = rows_bufs[slot]

        def tt_body(tt, _):
            p = [posc_v[tt, pl.ds(LANES * j, LANES)] for j in range(DJ)]
            for bb in range(GB):
                r = bb * TPOS + tt
                for j in range(DJ):
                    plsc.addupdate(rows_v.at[r, pl.ds(LANES * j, LANES)], p[j])
            return 0

        lax.fori_loop(0, TPOS, tt_body, 0)

    def fire_writes(t, slot):
        ci = t >> 2
        g = t & (NGROUP - 1)
        c = ci * TPOS
        rows_v = rows_bufs[slot]
        for bb in range(GB):
            grow = base_tok + (g * GB + bb) * MAX_LEN + c
            pltpu.async_copy(rows_v.at[pl.ds(bb * TPOS, TPOS), :],
                             out_hbm.at[pl.ds(grow, TPOS), :], w_sems[slot])

    def do_tile(t, slot, drain_cond, launch_cond):
        ci = t >> 2
        g = t & (NGROUP - 1)

        @pl.when(g == 0)
        def _():
            pltpu.sync_copy(pos_hbm.at[pl.ds(ci * TPOS, TPOS), :], posc_v)

        wait_gather(slot)
        nxt = (slot + 2) % NBUF
        if drain_cond is None:
            drain_writes(nxt)
        else:
            @pl.when(drain_cond)
            def _():
                drain_writes(nxt)
        if launch_cond is None:
            launch_gather(t + 2, nxt)
        else:
            @pl.when(launch_cond)
            def _():
                launch_gather(t + 2, nxt)
        add_pos(slot)
        fire_writes(t, slot)

    launch_gather(0, 0)
    launch_gather(1, 1)

    def triple_body(m, _):
        t0 = m * 3
        do_tile(t0, 0, m > 0, None)
        do_tile(t0 + 1, 1, None, None)
        do_tile(t0 + 2, 2, None, m < (NTILE // 3) - 1)
        return 0

    lax.fori_loop(0, NTILE // 3, triple_body, 0)

    t_last = NTILE - 1
    wait_gather(t_last % NBUF)
    add_pos(t_last % NBUF)
    fire_writes(t_last, t_last % NBUF)
    drain_writes((t_last - 1) % NBUF)
    drain_writes(t_last % NBUF)


def kernel(x, table, start_token, end_token):
    pos = _positional_encoding()
    table_rep = jnp.tile(table, (NW, 1))
    out = _sc_embed(x.reshape(-1), table_rep, pos)
    return out.reshape(BATCH, MAX_LEN, D_MODEL)


# speedup vs baseline: 1.4447x; 1.4447x over previous
import functools

import jax
import jax.numpy as jnp
from jax import lax
from jax.experimental import pallas as pl
from jax.experimental.pallas import tpu as pltpu, tpu_sc as plsc

D_MODEL = 512
MAX_LEN = 200
VOCAB = 100
BATCH = 1024

NC, NS, LANES = 2, 16, 16
NW = NC * NS
ROWS_PER_W = BATCH // NW
TOK_PER_W = ROWS_PER_W * MAX_LEN
TPOS = 8
GB = 8
TILE = TPOS * GB
NCHUNK = MAX_LEN // TPOS
NGROUP = ROWS_PER_W // GB
NTILE = NCHUNK * NGROUP
DJ = D_MODEL // LANES
NBUF = 3


def _positional_encoding():
    even_i = jnp.arange(0, D_MODEL, 2).astype(jnp.float32)
    denominator = jnp.power(10000.0, even_i / D_MODEL)
    position = jnp.arange(MAX_LEN, dtype=jnp.float32).reshape(MAX_LEN, 1)
    even = jnp.sin(position / denominator)
    odd = jnp.cos(position / denominator)
    stacked = jnp.stack([even, odd], axis=2)
    return stacked.reshape(MAX_LEN, D_MODEL)


@functools.partial(
    pl.kernel,
    mesh=plsc.VectorSubcoreMesh(core_axis_name="c", subcore_axis_name="s"),
    out_type=jax.ShapeDtypeStruct((BATCH * MAX_LEN, D_MODEL), jnp.float32),
    scratch_types=[
        pltpu.VMEM((TOK_PER_W,), jnp.int32),
        pltpu.VMEM((TPOS, D_MODEL), jnp.float32),
        pltpu.VMEM((NBUF, TILE), jnp.int32),
        pltpu.VMEM((TILE, D_MODEL), jnp.float32),
        pltpu.VMEM((TILE, D_MODEL), jnp.float32),
        pltpu.VMEM((TILE, D_MODEL), jnp.float32),
        pltpu.SemaphoreType.DMA,
        pltpu.SemaphoreType.DMA,
        pltpu.SemaphoreType.DMA,
        pltpu.SemaphoreType.DMA,
        pltpu.SemaphoreType.DMA,
        pltpu.SemaphoreType.DMA,
    ],
    compiler_params=pltpu.CompilerParams(needs_layout_passes=False),
)
def _sc_embed(x_hbm, table_hbm, pos_hbm, out_hbm,
              xw_v, posc_v, idx_v, rows0_v, rows1_v, rows2_v,
              g_sem0, g_sem1, g_sem2, w_sem0, w_sem1, w_sem2):
    wid = lax.axis_index("s") * NC + lax.axis_index("c")
    base_tok = wid * TOK_PER_W
    pltpu.sync_copy(x_hbm.at[pl.ds(base_tok, TOK_PER_W)], xw_v)
    lane = lax.iota(jnp.int32, LANES)

    rows_bufs = (rows0_v, rows1_v, rows2_v)
    g_sems = (g_sem0, g_sem1, g_sem2)
    w_sems = (w_sem0, w_sem1, w_sem2)

    def launch_gather(t, slot):
        ci = t >> 2
        g = t & (NGROUP - 1)
        c = ci * TPOS
        for k in range(TILE // LANES):
            gl = lane + (LANES * k)
            bb = gl >> 3
            tt = gl & (TPOS - 1)
            src = (g * GB + bb) * MAX_LEN + c + tt
            idx_v[slot, pl.ds(LANES * k, LANES)] = (
                plsc.load_gather(xw_v, [src]) + wid * VOCAB)
        pltpu.async_copy(table_hbm.at[idx_v.at[slot]], rows_bufs[slot],
                         g_sems[slot])

    def wait_gather(slot):
        pltpu.make_async_copy(out_hbm.at[pl.ds(0, TILE), :], rows_bufs[slot],
                              g_sems[slot]).wait()

    def drain_writes(slot):
        pltpu.make_async_copy(rows_bufs[slot], out_hbm.at[pl.ds(0, TILE), :],
                              w_sems[slot]).wait()

    def add_pos(slot):
        rows_v = rows_bufs[slot]

        def tt_body(tt, _):
            p = [posc_v[tt, pl.ds(LANES * j, LANES)] for j in range(DJ)]
            for bb in range(GB):
                r = bb * TPOS + tt
                for j in range(DJ):
                    plsc.addupdate(rows_v.at[r, pl.ds(LANES * j, LANES)], p[j])
            return 0

        lax.fori_loop(0, TPOS, tt_body, 0)

    def fire_writes(t, slot):
        ci = t >> 2
        g = t & (NGROUP - 1)
        c = ci * TPOS
        rows_v = rows_bufs[slot]
        for bb in range(GB):
            grow = base_tok + (g * GB + bb) * MAX_LEN + c
            pltpu.async_copy(rows_v.at[pl.ds(bb * TPOS, TPOS), :],
                             out_hbm.at[pl.ds(grow, TPOS), :], w_sems[slot])

    def do_tile(t, slot, drain_cond, launch_cond):
        ci = t >> 2
        g = t & (NGROUP - 1)

        @pl.when(g == 0)
        def _():
            pltpu.sync_copy(pos_hbm.at[pl.ds(ci * TPOS, TPOS), :], posc_v)

        wait_gather(slot)
        add_pos(slot)
        fire_writes(t, slot)
        nxt = (slot + 2) % NBUF
        if drain_cond is None:
            drain_writes(nxt)
        else:
            @pl.when(drain_cond)
            def _():
                drain_writes(nxt)
        if launch_cond is None:
            launch_gather(t + 2, nxt)
        else:
            @pl.when(launch_cond)
            def _():
                launch_gather(t + 2, nxt)

    launch_gather(0, 0)
    launch_gather(1, 1)

    def triple_body(m, _):
        t0 = m * 3
        do_tile(t0, 0, m > 0, None)
        do_tile(t0 + 1, 1, None, None)
        do_tile(t0 + 2, 2, None, m < (NTILE // 3) - 1)
        return 0

    lax.fori_loop(0, NTILE // 3, triple_body, 0)

    t_last = NTILE - 1
    wait_gather(t_last % NBUF)
    add_pos(t_last % NBUF)
    fire_writes(t_last, t_last % NBUF)
    drain_writes((t_last - 1) % NBUF)
    drain_writes(t_last % NBUF)


BR = 8


def _tc_body(x_ref, table_ref, pos_ref, out_ref):
    ids = x_ref[...]
    iota = lax.broadcasted_iota(jnp.int32, (BR * MAX_LEN, VOCAB), 1)
    onehot = (iota == ids).astype(jnp.float32)
    emb = jnp.dot(onehot, table_ref[...], preferred_element_type=jnp.float32)
    out_ref[...] = emb + pos_ref[...]


def _tc_embed(x2d, table, pos_rep):
    ntok = x2d.shape[0]
    blk = BR * MAX_LEN
    return pl.pallas_call(
        _tc_body,
        grid=(ntok // blk,),
        in_specs=[
            pl.BlockSpec((blk, 1), lambda i: (i, 0)),
            pl.BlockSpec((VOCAB, D_MODEL), lambda i: (0, 0)),
            pl.BlockSpec((blk, D_MODEL), lambda i: (0, 0)),
        ],
        out_specs=pl.BlockSpec((blk, D_MODEL), lambda i: (i, 0)),
        out_shape=jax.ShapeDtypeStruct((ntok, D_MODEL), jnp.float32),
    )(x2d, table, pos_rep)


def kernel(x, table, start_token, end_token):
    pos = _positional_encoding()
    pos_rep = jnp.tile(pos, (BR, 1))
    out = _tc_embed(x.reshape(-1, 1), table, pos_rep)
    return out.reshape(BATCH, MAX_LEN, D_MODEL)
